# baseline (speedup 1.0000x reference)
import jax
import jax.numpy as jnp
from jax import lax
from jax.experimental import pallas as pl
from jax.experimental.pallas import tpu as pltpu

N_E = 1024
D = 64
NTOK = 16 * 1024
BLK = 1024
NBLK = NTOK // BLK


def _vq_tc_body(x_ref, xn_ref, cb_ref, cn_ref, q_ref, loss_ref, ppl_ref,
                hist_ref, loss_s):
    i = pl.program_id(0)

    @pl.when(i == 0)
    def _init():
        hist_ref[...] = jnp.zeros_like(hist_ref)
        loss_s[0] = jnp.float32(0.0)

    x = x_ref[...]
    cb = cb_ref[...]
    s2 = jnp.dot(x * jnp.float32(-2.0), cb,
                 preferred_element_type=jnp.float32)
    dist = (xn_ref[...] + cn_ref[...]) + s2
    m = jnp.min(dist, axis=1, keepdims=True)
    lane_f = lax.broadcasted_iota(jnp.int32, (1, N_E), 1).astype(jnp.float32)
    idxs_f = jnp.min(jnp.where(dist == m, lane_f, jnp.float32(N_E)), axis=1)

    oh = (lane_f == idxs_f[:, None]).astype(jnp.float32)
    hist_ref[...] += jnp.sum(oh, axis=0, keepdims=True)
    q_ref[...] = lax.dot_general(oh, cb, (((1,), (1,)), ((), ())),
                                 preferred_element_type=jnp.float32)
    loss_s[0] += jnp.sum(m)

    @pl.when(i == NBLK - 1)
    def _finish():
        loss_ref[0] = loss_s[0] * jnp.float32(1.0 / (NTOK * D))
        p = hist_ref[...] * jnp.float32(1.0 / NTOK)
        ent = jnp.sum(p * jnp.log(p + jnp.float32(1e-10)))
        ppl_ref[0] = jnp.exp(-ent)


def _vq_call(flatten, xn, codebook, cn):
    return pl.pallas_call(
        _vq_tc_body,
        grid=(NBLK,),
        in_specs=[
            pl.BlockSpec((BLK, D), lambda i: (i, 0)),
            pl.BlockSpec((BLK, 1), lambda i: (i, 0)),
            pl.BlockSpec((D, N_E), lambda i: (0, 0)),
            pl.BlockSpec((1, N_E), lambda i: (0, 0)),
        ],
        out_specs=[
            pl.BlockSpec((BLK, D), lambda i: (i, 0)),
            pl.BlockSpec(memory_space=pltpu.SMEM),
            pl.BlockSpec(memory_space=pltpu.SMEM),
        ],
        out_shape=[
            jax.ShapeDtypeStruct((NTOK, D), jnp.float32),
            jax.ShapeDtypeStruct((1,), jnp.float32),
            jax.ShapeDtypeStruct((1,), jnp.float32),
        ],
        scratch_shapes=[
            pltpu.VMEM((1, N_E), jnp.float32),
            pltpu.SMEM((1,), jnp.float32),
        ],
        compiler_params=pltpu.CompilerParams(
            dimension_semantics=("arbitrary",),
        ),
    )(flatten, xn, codebook, cn)


def kernel(inputs, codebook):
    flatten = inputs.reshape(NTOK, D)
    xn = jnp.sum(flatten ** 2.0, axis=1, keepdims=True)
    cn = jnp.sum(codebook ** 2.0, axis=0, keepdims=True)
    q, loss, ppl = _vq_call(flatten, xn, codebook, cn)
    quantized = q.reshape(inputs.shape)
    return (quantized, loss[0], ppl[0])


# speedup vs baseline: 1.0990x; 1.0990x over previous
import jax
import jax.numpy as jnp
from jax import lax
from jax.experimental import pallas as pl
from jax.experimental.pallas import tpu as pltpu

N_E = 1024
D = 64
NTOK = 16 * 1024
BLK = 2048
NBLK = NTOK // BLK


def _vq_tc_body(x_ref, cb_ref, idx_ref, q_ref, loss_ref, ppl_ref,
                hist_ref, loss_s):
    i = pl.program_id(0)

    @pl.when(i == 0)
    def _init():
        hist_ref[...] = jnp.zeros_like(hist_ref)
        loss_s[0] = jnp.float32(0.0)

    x = x_ref[...]
    cb = cb_ref[...]
    s2 = jnp.dot(x * jnp.float32(-2.0), cb,
                 preferred_element_type=jnp.float32)
    xn = jnp.sum(x * x, axis=1, keepdims=True)
    cn = jnp.sum(cb * cb, axis=0, keepdims=True)
    dist = (xn + cn) + s2
    m = jnp.min(dist, axis=1, keepdims=True)
    lane_f = lax.broadcasted_iota(jnp.int32, (1, N_E), 1).astype(jnp.float32)
    idxs_f = jnp.min(jnp.where(dist == m, lane_f, jnp.float32(N_E)), axis=1)
    idx_ref[0, 0, :] = idxs_f.astype(jnp.int32)

    oh = (lane_f == idxs_f[:, None]).astype(jnp.float32)
    hist_ref[...] += jnp.sum(oh, axis=0, keepdims=True)
    q_ref[...] = lax.dot_general(oh, cb, (((1,), (1,)), ((), ())),
                                 preferred_element_type=jnp.float32)
    loss_s[0] += jnp.sum(m)

    @pl.when(i == NBLK - 1)
    def _finish():
        loss_ref[0] = loss_s[0] * jnp.float32(1.0 / (NTOK * D))
        p = hist_ref[...] * jnp.float32(1.0 / NTOK)
        ent = jnp.sum(p * jnp.log(p + jnp.float32(1e-10)))
        ppl_ref[0] = jnp.exp(-ent)


def _tc_stats(flatten, codebook):
    return pl.pallas_call(
        _vq_tc_body,
        grid=(NBLK,),
        in_specs=[
            pl.BlockSpec((BLK, D), lambda i: (i, 0)),
            pl.BlockSpec((D, N_E), lambda i: (0, 0)),
        ],
        out_specs=[
            pl.BlockSpec((1, 1, BLK), lambda i: (i, 0, 0)),
            pl.BlockSpec((BLK, D), lambda i: (i, 0)),
            pl.BlockSpec(memory_space=pltpu.SMEM),
            pl.BlockSpec(memory_space=pltpu.SMEM),
        ],
        out_shape=[
            jax.ShapeDtypeStruct((NBLK, 1, BLK), jnp.int32),
            jax.ShapeDtypeStruct((NTOK, D), jnp.float32),
            jax.ShapeDtypeStruct((1,), jnp.float32),
            jax.ShapeDtypeStruct((1,), jnp.float32),
        ],
        scratch_shapes=[
            pltpu.VMEM((1, N_E), jnp.float32),
            pltpu.SMEM((1,), jnp.float32),
        ],
        compiler_params=pltpu.CompilerParams(
            dimension_semantics=("arbitrary",),
        ),
    )(flatten, codebook)


def kernel(inputs, codebook):
    flatten = inputs.reshape(NTOK, D)
    idx3, q, loss, ppl = _tc_stats(flatten, codebook)
    quantized = q.reshape(inputs.shape)
    return (quantized, loss[0], ppl[0])


# speedup vs baseline: 1.1223x; 1.0212x over previous
import jax
import jax.numpy as jnp
from jax import lax
from jax.experimental import pallas as pl
from jax.experimental.pallas import tpu as pltpu

N_E = 1024
D = 64
NTOK = 16 * 1024
BLK = 4096
NBLK = NTOK // BLK


def _vq_tc_body(x_ref, cb_ref, idx_ref, q_ref, loss_ref, ppl_ref,
                hist_ref, loss_s):
    i = pl.program_id(0)

    @pl.when(i == 0)
    def _init():
        hist_ref[...] = jnp.zeros_like(hist_ref)
        loss_s[0] = jnp.float32(0.0)

    x = x_ref[...]
    cb = cb_ref[...]
    s2 = jnp.dot(x * jnp.float32(-2.0), cb,
                 preferred_element_type=jnp.float32)
    xn = jnp.sum(x * x, axis=1, keepdims=True)
    cn = jnp.sum(cb * cb, axis=0, keepdims=True)
    dist = (xn + cn) + s2
    m = jnp.min(dist, axis=1, keepdims=True)
    lane_f = lax.broadcasted_iota(jnp.int32, (1, N_E), 1).astype(jnp.float32)
    idxs_f = jnp.min(jnp.where(dist == m, lane_f, jnp.float32(N_E)), axis=1)
    idx_ref[0, 0, :] = idxs_f.astype(jnp.int32)

    oh = (lane_f == idxs_f[:, None]).astype(jnp.float32)
    hist_ref[...] += jnp.sum(oh, axis=0, keepdims=True)
    q_ref[...] = lax.dot_general(oh, cb, (((1,), (1,)), ((), ())),
                                 preferred_element_type=jnp.float32)
    loss_s[0] += jnp.sum(m)

    @pl.when(i == NBLK - 1)
    def _finish():
        loss_ref[0] = loss_s[0] * jnp.float32(1.0 / (NTOK * D))
        p = hist_ref[...] * jnp.float32(1.0 / NTOK)
        ent = jnp.sum(p * jnp.log(p + jnp.float32(1e-10)))
        ppl_ref[0] = jnp.exp(-ent)


def _tc_stats(flatten, codebook):
    return pl.pallas_call(
        _vq_tc_body,
        grid=(NBLK,),
        in_specs=[
            pl.BlockSpec((BLK, D), lambda i: (i, 0)),
            pl.BlockSpec((D, N_E), lambda i: (0, 0)),
        ],
        out_specs=[
            pl.BlockSpec((1, 1, BLK), lambda i: (i, 0, 0)),
            pl.BlockSpec((BLK, D), lambda i: (i, 0)),
            pl.BlockSpec(memory_space=pltpu.SMEM),
            pl.BlockSpec(memory_space=pltpu.SMEM),
        ],
        out_shape=[
            jax.ShapeDtypeStruct((NBLK, 1, BLK), jnp.int32),
            jax.ShapeDtypeStruct((NTOK, D), jnp.float32),
            jax.ShapeDtypeStruct((1,), jnp.float32),
            jax.ShapeDtypeStruct((1,), jnp.float32),
        ],
        scratch_shapes=[
            pltpu.VMEM((1, N_E), jnp.float32),
            pltpu.SMEM((1,), jnp.float32),
        ],
        compiler_params=pltpu.CompilerParams(
            dimension_semantics=("arbitrary",),
        ),
    )(flatten, codebook)


def kernel(inputs, codebook):
    flatten = inputs.reshape(NTOK, D)
    idx3, q, loss, ppl = _tc_stats(flatten, codebook)
    quantized = q.reshape(inputs.shape)
    return (quantized, loss[0], ppl[0])
